# baseline (speedup 1.0000x reference)
import functools

import jax
import jax.numpy as jnp
from jax import lax
from jax.experimental import pallas as pl
from jax.experimental.pallas import tpu as pltpu
from jax.experimental.pallas import tpu_sc as plsc

NUM_FEATURES = 26
VOCAB = 100000
EMBED_DIM = 32
BATCH = 16384

NUM_CORES = 2
NUM_SUBCORES = 16
NUM_WORKERS = NUM_CORES * NUM_SUBCORES
BPW = BATCH // NUM_WORKERS
ICHUNK = 128
NCHUNK = BPW // ICHUNK


def _sc_body(inputs_hbm, tables_hbm, out_hbm, idxblk_v, idxf_v, rows_v, gsem):
    wid = lax.axis_index("s") * NUM_CORES + lax.axis_index("c")
    base = wid * BPW

    pltpu.sync_copy(inputs_hbm.at[pl.ds(base * NUM_FEATURES, BPW * NUM_FEATURES)], idxblk_v)

    def feat_body(f, carry):
        off = f * VOCAB

        def jbody(j, c2):
            r0 = j * 16
            ridx = (r0 + lax.iota(jnp.int32, 16)) * NUM_FEATURES + f
            vals = plsc.load_gather(idxblk_v, [ridx])
            idxf_v[j // 8, pl.ds((j % 8) * 16, 16)] = vals + off
            return c2

        lax.fori_loop(0, BPW // 16, jbody, 0, unroll=True)

        copies = []
        for c in range(NCHUNK):
            copies.append(
                pltpu.async_copy(
                    tables_hbm.at[idxf_v.at[c]],
                    rows_v.at[pl.ds(c * ICHUNK, ICHUNK)],
                    gsem,
                )
            )
        for cp in copies:
            cp.wait()

        pltpu.sync_copy(rows_v, out_hbm.at[pl.ds(f * BATCH + base, BPW)])
        return carry

    lax.fori_loop(0, NUM_FEATURES, feat_body, 0)


@jax.jit
def kernel(inputs, tables):
    tables_flat = tables.reshape(NUM_FEATURES * VOCAB, EMBED_DIM)
    inputs_flat = inputs.reshape(BATCH * NUM_FEATURES)
    run = pl.kernel(
        _sc_body,
        out_type=jax.ShapeDtypeStruct((NUM_FEATURES * BATCH, EMBED_DIM), jnp.float32),
        mesh=plsc.VectorSubcoreMesh(core_axis_name="c", subcore_axis_name="s"),
        compiler_params=pltpu.CompilerParams(
            needs_layout_passes=False, use_tc_tiling_on_sc=False
        ),
        scratch_types=[
            pltpu.VMEM((BPW * NUM_FEATURES,), jnp.int32),
            pltpu.VMEM((NCHUNK, ICHUNK), jnp.int32),
            pltpu.VMEM((BPW, EMBED_DIM), jnp.float32),
            pltpu.SemaphoreType.DMA,
        ],
    )
    out_flat = run(inputs_flat, tables_flat)
    return out_flat.reshape(NUM_FEATURES, BATCH, EMBED_DIM)


# speedup vs baseline: 3.7124x; 3.7124x over previous
import jax
import jax.numpy as jnp
from jax import lax
from jax.experimental import pallas as pl
from jax.experimental.pallas import tpu as pltpu
from jax.experimental.pallas import tpu_sc as plsc

NUM_FEATURES = 26
VOCAB = 100000
EMBED_DIM = 32
BATCH = 16384

NUM_CORES = 2
NUM_SUBCORES = 16
NUM_WORKERS = NUM_CORES * NUM_SUBCORES

CHUNK = 8192
NCH = BATCH // CHUNK


def _sc_body(idx_hbm, tab_hbm, out_hbm, row_v, idx_v, out_v, sem):
    wid = lax.axis_index("s") * NUM_CORES + lax.axis_index("c")
    r = wid

    for f in range(NUM_FEATURES):
        pltpu.sync_copy(tab_hbm.at[f, r], row_v)
        for c in range(NCH):
            pltpu.sync_copy(idx_hbm.at[f, pl.ds(c * CHUNK, CHUNK)], idx_v)

            def jbody(j, carry):
                iv = idx_v[pl.ds(j * 16, 16)]
                out_v[pl.ds(j * 16, 16)] = plsc.load_gather(row_v, [iv])
                return carry

            lax.fori_loop(0, CHUNK // 16, jbody, 0)
            pltpu.sync_copy(out_v, out_hbm.at[f, r, pl.ds(c * CHUNK, CHUNK)])


@jax.jit
def kernel(inputs, tables):
    tables_t = tables.transpose(0, 2, 1)
    inputs_t = inputs.T.astype(jnp.int32)
    run = pl.kernel(
        _sc_body,
        out_type=jax.ShapeDtypeStruct((NUM_FEATURES, EMBED_DIM, BATCH), jnp.float32),
        mesh=plsc.VectorSubcoreMesh(core_axis_name="c", subcore_axis_name="s"),
        compiler_params=pltpu.CompilerParams(needs_layout_passes=False),
        scratch_types=[
            pltpu.VMEM((VOCAB,), jnp.float32),
            pltpu.VMEM((CHUNK,), jnp.int32),
            pltpu.VMEM((CHUNK,), jnp.float32),
            pltpu.SemaphoreType.DMA,
        ],
    )
    out_t = run(inputs_t, tables_t)
    return out_t.transpose(0, 2, 1)
